# baseline (speedup 1.0000x reference)
import functools

import jax
import jax.numpy as jnp
from jax import lax
from jax.experimental import pallas as pl
from jax.experimental.pallas import tpu as pltpu
from jax.experimental.pallas import tpu_sc as plsc

SEQ_LEN = 200
VOCAB = 100000
D = 64
BATCH = 4096

NUM_WORKERS = 32
SEQ_PER_WORKER = BATCH // NUM_WORKERS
CHUNK_SEQ = 2
CHUNK_ROWS = CHUNK_SEQ * SEQ_LEN
OUT_CHUNK = CHUNK_ROWS // 2
NUM_CHUNKS = SEQ_PER_WORKER // CHUNK_SEQ
GATHER_SPLIT = 5
GATHER_ROWS = CHUNK_ROWS // GATHER_SPLIT
ROWS_PER_WORKER = SEQ_PER_WORKER * SEQ_LEN


def _sc_body(idx_hbm, table_hbm, pos_hbm, out_hbm, idx_v, g_v, s_v, pos_v,
             gsem, osem):
    wid = lax.axis_index("s") * 2 + lax.axis_index("c")
    worker_base = wid * ROWS_PER_WORKER

    pltpu.sync_copy(pos_hbm, pos_v)

    def fire_chunk(c, p):
        row_base = worker_base + c * CHUNK_ROWS
        pltpu.sync_copy(idx_hbm.at[pl.ds(row_base, CHUNK_ROWS)],
                        idx_v.at[pl.ds(p * CHUNK_ROWS, CHUNK_ROWS)])
        for j in range(GATHER_SPLIT):
            off = p * CHUNK_ROWS + j * GATHER_ROWS
            pltpu.async_copy(
                table_hbm.at[idx_v.at[pl.ds(off, GATHER_ROWS)]],
                g_v.at[pl.ds(off, GATHER_ROWS)],
                gsem,
            )

    fire_chunk(0, 0)

    def chunk_body(c, carry):
        p = lax.rem(c, 2)
        pn = 1 - p
        gbase = p * CHUNK_ROWS
        sbase = p * OUT_CHUNK

        @pl.when(c > 1)
        def _():
            pltpu.make_async_copy(
                s_v.at[pl.ds(sbase, OUT_CHUNK)],
                out_hbm.at[pl.ds(0, OUT_CHUNK)],
                osem).wait()

        @pl.when(c < NUM_CHUNKS - 1)
        def _():
            fire_chunk(c + 1, pn)

        pltpu.make_async_copy(
            out_hbm.at[pl.ds(0, OUT_CHUNK)],
            g_v.at[pl.ds(gbase, CHUNK_ROWS)],
            gsem).wait()

        def add_body(m, carry2):
            for u in range(2):
                for r in range(D // 16):
                    pv = pos_v[pl.ds(m * 2 * D + u * D + r * 16, 16)]
                    for s in range(CHUNK_SEQ):
                        grow = gbase + s * SEQ_LEN + 2 * m + u
                        srow = sbase + s * (SEQ_LEN // 2) + m
                        s_v[srow, pl.ds(u * D + r * 16, 16)] = (
                            g_v[grow, pl.ds(r * 16, 16)] + pv)
            return carry2

        lax.fori_loop(0, SEQ_LEN // 2, add_body, 0)

        out_base = (worker_base + c * CHUNK_ROWS) // 2
        pltpu.async_copy(
            s_v.at[pl.ds(sbase, OUT_CHUNK)],
            out_hbm.at[pl.ds(out_base, OUT_CHUNK)],
            osem)
        return carry

    lax.fori_loop(0, NUM_CHUNKS, chunk_body, 0)

    for p in (0, 1):
        pltpu.make_async_copy(
            s_v.at[pl.ds(p * OUT_CHUNK, OUT_CHUNK)],
            out_hbm.at[pl.ds(0, OUT_CHUNK)],
            osem).wait()


def _sc_gather_add(flat_idx, word_table, pos_flat):
    mesh = plsc.VectorSubcoreMesh(core_axis_name="c", subcore_axis_name="s")
    return pl.kernel(
        _sc_body,
        mesh=mesh,
        compiler_params=pltpu.CompilerParams(use_tc_tiling_on_sc=False),
        out_type=jax.ShapeDtypeStruct((BATCH * SEQ_LEN // 2, 2 * D), jnp.float32),
        scratch_types=[
            pltpu.VMEM((2 * CHUNK_ROWS,), jnp.int32),
            pltpu.VMEM((2 * CHUNK_ROWS, D), jnp.float32),
            pltpu.VMEM((2 * OUT_CHUNK, 2 * D), jnp.float32),
            pltpu.VMEM((SEQ_LEN * D,), jnp.float32),
            pltpu.SemaphoreType.DMA,
            pltpu.SemaphoreType.DMA,
        ],
    )(flat_idx, word_table, pos_flat)


RB = 2048


def _tc_relayout_body(x_ref, o_ref):
    a = x_ref[...]
    lo = a[:, :D]
    hi = a[:, D:]
    o_ref[...] = jnp.stack([lo, hi], axis=1).reshape(2 * RB, D)


def _tc_relayout(x):
    return pl.pallas_call(
        _tc_relayout_body,
        grid=(BATCH * SEQ_LEN // 2 // RB,),
        in_specs=[pl.BlockSpec((RB, 2 * D), lambda i: (i, 0))],
        out_specs=pl.BlockSpec((2 * RB, D), lambda i: (i, 0)),
        out_shape=jax.ShapeDtypeStruct((BATCH * SEQ_LEN, D), jnp.float32),
    )(x)


@jax.jit
def _pos_embed(flat_idx, word_table, pos_flat):
    dense = _sc_gather_add(flat_idx, word_table, pos_flat)
    return _tc_relayout(dense)


def kernel(inputs, word_table, pos_table):
    flat_idx = inputs.reshape(-1)
    pos_flat = pos_table.reshape(-1)
    out = _pos_embed(flat_idx, word_table, pos_flat)
    return out.reshape(BATCH, SEQ_LEN, D)


# speedup vs baseline: 1.1146x; 1.1146x over previous
import jax
import jax.numpy as jnp
from jax import lax
from jax.experimental import pallas as pl
from jax.experimental.pallas import tpu as pltpu
from jax.experimental.pallas import tpu_sc as plsc

SEQ_LEN = 200
VOCAB = 100000
D = 64
BATCH = 4096

NUM_WORKERS = 32
SEQ_PER_WORKER = BATCH // NUM_WORKERS
CHUNK_SEQ = 2
CHUNK_ROWS = CHUNK_SEQ * SEQ_LEN
NUM_CHUNKS = SEQ_PER_WORKER // CHUNK_SEQ
GATHER_SPLIT = 5
GATHER_ROWS = CHUNK_ROWS // GATHER_SPLIT
ROWS_PER_WORKER = SEQ_PER_WORKER * SEQ_LEN


def _sc_body(idx_hbm, table_hbm, pos_hbm, out_hbm, idx_v, g_v, s_v, pos_v,
             gsem, osem):
    wid = lax.axis_index("s") * 2 + lax.axis_index("c")
    worker_base = wid * ROWS_PER_WORKER

    pltpu.sync_copy(pos_hbm, pos_v)

    def fire_chunk(c, p):
        row_base = worker_base + c * CHUNK_ROWS
        pltpu.sync_copy(idx_hbm.at[pl.ds(row_base, CHUNK_ROWS)],
                        idx_v.at[pl.ds(p * CHUNK_ROWS, CHUNK_ROWS)])
        for j in range(GATHER_SPLIT):
            off = p * CHUNK_ROWS + j * GATHER_ROWS
            pltpu.async_copy(
                table_hbm.at[idx_v.at[pl.ds(off, GATHER_ROWS)]],
                g_v.at[pl.ds(off, GATHER_ROWS)],
                gsem,
            )

    fire_chunk(0, 0)

    def chunk_body(c, carry):
        p = lax.rem(c, 2)
        pn = 1 - p
        gbase = p * CHUNK_ROWS

        @pl.when(c > 1)
        def _():
            pltpu.make_async_copy(
                s_v.at[pl.ds(gbase, CHUNK_ROWS)],
                out_hbm.at[pl.ds(0, CHUNK_ROWS)],
                osem).wait()

        @pl.when(c < NUM_CHUNKS - 1)
        def _():
            fire_chunk(c + 1, pn)

        pltpu.make_async_copy(
            out_hbm.at[pl.ds(0, CHUNK_ROWS)],
            g_v.at[pl.ds(gbase, CHUNK_ROWS)],
            gsem).wait()

        def add_body(l, carry2):
            for r in range(D // 16):
                pv = pos_v[pl.ds(l * D + r * 16, 16)]
                for s in range(CHUNK_SEQ):
                    row = gbase + s * SEQ_LEN + l
                    s_v[row, pl.ds(r * 16, 16)] = (
                        g_v[row, pl.ds(r * 16, 16)] + pv)
            return carry2

        lax.fori_loop(0, SEQ_LEN, add_body, 0)

        out_base = worker_base + c * CHUNK_ROWS
        pltpu.async_copy(
            s_v.at[pl.ds(gbase, CHUNK_ROWS)],
            out_hbm.at[pl.ds(out_base, CHUNK_ROWS)],
            osem)
        return carry

    lax.fori_loop(0, NUM_CHUNKS, chunk_body, 0)

    for p in (0, 1):
        pltpu.make_async_copy(
            s_v.at[pl.ds(p * CHUNK_ROWS, CHUNK_ROWS)],
            out_hbm.at[pl.ds(0, CHUNK_ROWS)],
            osem).wait()


def _sc_gather_add(flat_idx, word_table, pos_flat):
    mesh = plsc.VectorSubcoreMesh(core_axis_name="c", subcore_axis_name="s")
    return pl.kernel(
        _sc_body,
        mesh=mesh,
        compiler_params=pltpu.CompilerParams(use_tc_tiling_on_sc=False),
        out_type=jax.ShapeDtypeStruct((BATCH * SEQ_LEN, D), jnp.float32),
        scratch_types=[
            pltpu.VMEM((2 * CHUNK_ROWS,), jnp.int32),
            pltpu.VMEM((2 * CHUNK_ROWS, D), jnp.float32),
            pltpu.VMEM((2 * CHUNK_ROWS, D), jnp.float32),
            pltpu.VMEM((SEQ_LEN * D,), jnp.float32),
            pltpu.SemaphoreType.DMA,
            pltpu.SemaphoreType.DMA,
        ],
    )(flat_idx, word_table, pos_flat)


@jax.jit
def _pos_embed(flat_idx, word_table, pos_flat):
    return _sc_gather_add(flat_idx, word_table, pos_flat)


def kernel(inputs, word_table, pos_table):
    flat_idx = inputs.reshape(-1)
    pos_flat = pos_table.reshape(-1)
    out = _pos_embed(flat_idx, word_table, pos_flat)
    return out.reshape(BATCH, SEQ_LEN, D)


# speedup vs baseline: 1.5683x; 1.4071x over previous
import jax
import jax.numpy as jnp
from jax import lax
from jax.experimental import pallas as pl
from jax.experimental.pallas import tpu as pltpu
from jax.experimental.pallas import tpu_sc as plsc

SEQ_LEN = 200
VOCAB = 100000
D = 64
BATCH = 4096

NUM_WORKERS = 32
SEQ_PER_WORKER = BATCH // NUM_WORKERS
CHUNK_SEQ = 4
CHUNK_ROWS = CHUNK_SEQ * SEQ_LEN
NUM_CHUNKS = SEQ_PER_WORKER // CHUNK_SEQ
GATHER_ROWS = 80
GATHER_SPLIT = CHUNK_ROWS // GATHER_ROWS
ROWS_PER_WORKER = SEQ_PER_WORKER * SEQ_LEN


def _sc_body(idx_hbm, table_hbm, pos_hbm, out_hbm, idx_v, g_v, pos_v, gsem):
    wid = lax.axis_index("s") * 2 + lax.axis_index("c")
    worker_base = wid * ROWS_PER_WORKER

    pltpu.sync_copy(pos_hbm, pos_v)

    def chunk_body(c, carry):
        row_base = worker_base + c * CHUNK_ROWS

        pltpu.sync_copy(idx_hbm.at[pl.ds(row_base, CHUNK_ROWS)], idx_v)
        for j in range(GATHER_SPLIT):
            pltpu.async_copy(
                table_hbm.at[idx_v.at[pl.ds(j * GATHER_ROWS, GATHER_ROWS)]],
                g_v.at[pl.ds(j * GATHER_ROWS, GATHER_ROWS)],
                gsem,
            )
        pltpu.make_async_copy(
            out_hbm.at[pl.ds(0, CHUNK_ROWS)], g_v, gsem).wait()

        def add_body(l, carry2):
            for r in range(D // 16):
                pv = pos_v[pl.ds(l * D + r * 16, 16)]
                for s in range(CHUNK_SEQ):
                    row = s * SEQ_LEN + l
                    g_v[row, pl.ds(r * 16, 16)] = (
                        g_v[row, pl.ds(r * 16, 16)] + pv)
            return carry2

        lax.fori_loop(0, SEQ_LEN, add_body, 0)

        pltpu.sync_copy(g_v, out_hbm.at[pl.ds(row_base, CHUNK_ROWS)])
        return carry

    lax.fori_loop(0, NUM_CHUNKS, chunk_body, 0)


def _sc_gather_add(flat_idx, word_table, pos_flat):
    mesh = plsc.VectorSubcoreMesh(core_axis_name="c", subcore_axis_name="s")
    return pl.kernel(
        _sc_body,
        mesh=mesh,
        compiler_params=pltpu.CompilerParams(use_tc_tiling_on_sc=False),
        out_type=jax.ShapeDtypeStruct((BATCH * SEQ_LEN, D), jnp.float32),
        scratch_types=[
            pltpu.VMEM((CHUNK_ROWS,), jnp.int32),
            pltpu.VMEM((CHUNK_ROWS, D), jnp.float32),
            pltpu.VMEM((SEQ_LEN * D,), jnp.float32),
            pltpu.SemaphoreType.DMA,
        ],
    )(flat_idx, word_table, pos_flat)


@jax.jit
def _pos_embed(flat_idx, word_table, pos_flat):
    return _sc_gather_add(flat_idx, word_table, pos_flat)


def kernel(inputs, word_table, pos_table):
    flat_idx = inputs.reshape(-1)
    pos_flat = pos_table.reshape(-1)
    out = _pos_embed(flat_idx, word_table, pos_flat)
    return out.reshape(BATCH, SEQ_LEN, D)


# speedup vs baseline: 1.6289x; 1.0386x over previous
import jax
import jax.numpy as jnp
from jax import lax
from jax.experimental import pallas as pl
from jax.experimental.pallas import tpu as pltpu
from jax.experimental.pallas import tpu_sc as plsc

SEQ_LEN = 200
VOCAB = 100000
D = 64
BATCH = 4096

NUM_WORKERS = 32
SEQ_PER_WORKER = BATCH // NUM_WORKERS
CHUNK_SEQ = 8
CHUNK_ROWS = CHUNK_SEQ * SEQ_LEN
NUM_CHUNKS = SEQ_PER_WORKER // CHUNK_SEQ
GATHER_ROWS = 80
GATHER_SPLIT = CHUNK_ROWS // GATHER_ROWS
ROWS_PER_WORKER = SEQ_PER_WORKER * SEQ_LEN


def _sc_body(idx_hbm, table_hbm, pos_hbm, out_hbm, idx_v, g_v, pos_v, gsem):
    wid = lax.axis_index("s") * 2 + lax.axis_index("c")
    worker_base = wid * ROWS_PER_WORKER

    pltpu.sync_copy(pos_hbm, pos_v)

    def chunk_body(c, carry):
        row_base = worker_base + c * CHUNK_ROWS

        pltpu.sync_copy(idx_hbm.at[pl.ds(row_base, CHUNK_ROWS)], idx_v)
        for j in range(GATHER_SPLIT):
            pltpu.async_copy(
                table_hbm.at[idx_v.at[pl.ds(j * GATHER_ROWS, GATHER_ROWS)]],
                g_v.at[pl.ds(j * GATHER_ROWS, GATHER_ROWS)],
                gsem,
            )
        pltpu.make_async_copy(
            out_hbm.at[pl.ds(0, CHUNK_ROWS)], g_v, gsem).wait()

        def add_body(l, carry2):
            for r in range(D // 16):
                pv = pos_v[pl.ds(l * D + r * 16, 16)]
                for s in range(CHUNK_SEQ):
                    row = s * SEQ_LEN + l
                    g_v[row, pl.ds(r * 16, 16)] = (
                        g_v[row, pl.ds(r * 16, 16)] + pv)
            return carry2

        lax.fori_loop(0, SEQ_LEN, add_body, 0)

        pltpu.sync_copy(g_v, out_hbm.at[pl.ds(row_base, CHUNK_ROWS)])
        return carry

    lax.fori_loop(0, NUM_CHUNKS, chunk_body, 0)


def _sc_gather_add(flat_idx, word_table, pos_flat):
    mesh = plsc.VectorSubcoreMesh(core_axis_name="c", subcore_axis_name="s")
    return pl.kernel(
        _sc_body,
        mesh=mesh,
        compiler_params=pltpu.CompilerParams(use_tc_tiling_on_sc=False),
        out_type=jax.ShapeDtypeStruct((BATCH * SEQ_LEN, D), jnp.float32),
        scratch_types=[
            pltpu.VMEM((CHUNK_ROWS,), jnp.int32),
            pltpu.VMEM((CHUNK_ROWS, D), jnp.float32),
            pltpu.VMEM((SEQ_LEN * D,), jnp.float32),
            pltpu.SemaphoreType.DMA,
        ],
    )(flat_idx, word_table, pos_flat)


@jax.jit
def _pos_embed(flat_idx, word_table, pos_flat):
    return _sc_gather_add(flat_idx, word_table, pos_flat)


def kernel(inputs, word_table, pos_table):
    flat_idx = inputs.reshape(-1)
    pos_flat = pos_table.reshape(-1)
    out = _pos_embed(flat_idx, word_table, pos_flat)
    return out.reshape(BATCH, SEQ_LEN, D)


# speedup vs baseline: 1.7973x; 1.1033x over previous
import jax
import jax.numpy as jnp
from jax import lax
from jax.experimental import pallas as pl
from jax.experimental.pallas import tpu as pltpu
from jax.experimental.pallas import tpu_sc as plsc

SEQ_LEN = 200
VOCAB = 100000
D = 64
BATCH = 4096

NUM_WORKERS = 32
SEQ_PER_WORKER = BATCH // NUM_WORKERS
CHUNK_SEQ = 2
CHUNK_ROWS = CHUNK_SEQ * SEQ_LEN
NUM_CHUNKS = SEQ_PER_WORKER // CHUNK_SEQ
GATHER_ROWS = 80
GATHER_SPLIT = CHUNK_ROWS // GATHER_ROWS
ROWS_PER_WORKER = SEQ_PER_WORKER * SEQ_LEN
NBUF = 4


def _sc_body(idx_hbm, table_hbm, pos_hbm, out_hbm, idx_v, g_v, pos_v,
             gsem, osem, isem):
    wid = lax.axis_index("s") * 2 + lax.axis_index("c")
    worker_base = wid * ROWS_PER_WORKER

    pltpu.sync_copy(pos_hbm, pos_v)

    def fire_idx(c, b):
        pltpu.async_copy(
            idx_hbm.at[pl.ds(worker_base + c * CHUNK_ROWS, CHUNK_ROWS)],
            idx_v.at[pl.ds(b * CHUNK_ROWS, CHUNK_ROWS)],
            isem)

    def fire_gathers(c, b):
        pltpu.make_async_copy(
            idx_hbm.at[pl.ds(0, CHUNK_ROWS)],
            idx_v.at[pl.ds(b * CHUNK_ROWS, CHUNK_ROWS)],
            isem).wait()
        for j in range(GATHER_SPLIT):
            off = b * CHUNK_ROWS + j * GATHER_ROWS
            pltpu.async_copy(
                table_hbm.at[idx_v.at[pl.ds(off, GATHER_ROWS)]],
                g_v.at[pl.ds(off, GATHER_ROWS)],
                gsem)

    def wait_gathers(b):
        pltpu.make_async_copy(
            out_hbm.at[pl.ds(0, CHUNK_ROWS)],
            g_v.at[pl.ds(b * CHUNK_ROWS, CHUNK_ROWS)],
            gsem).wait()

    def wait_wb():
        pltpu.make_async_copy(
            g_v.at[pl.ds(0, CHUNK_ROWS)],
            out_hbm.at[pl.ds(0, CHUNK_ROWS)],
            osem).wait()

    def add_chunk(b):
        gbase = b * CHUNK_ROWS

        def add_body(l, carry):
            for r in range(D // 16):
                pv = pos_v[pl.ds(l * D + r * 16, 16)]
                for s in range(CHUNK_SEQ):
                    row = gbase + s * SEQ_LEN + l
                    g_v[row, pl.ds(r * 16, 16)] = (
                        g_v[row, pl.ds(r * 16, 16)] + pv)
            return carry

        lax.fori_loop(0, SEQ_LEN, add_body, 0)

    def fire_wb(c, b):
        pltpu.async_copy(
            g_v.at[pl.ds(b * CHUNK_ROWS, CHUNK_ROWS)],
            out_hbm.at[pl.ds(worker_base + c * CHUNK_ROWS, CHUNK_ROWS)],
            osem)

    def step(c, k, wait_w, fire_g, fire_i):
        b2 = (k + 2) % NBUF
        if wait_w:
            wait_wb()
        if fire_g:
            fire_gathers(c + 2, b2)
        wait_gathers(k)
        if fire_i:
            fire_idx(c + 4, k)
        add_chunk(k)
        fire_wb(c, k)

    for c0 in range(NBUF):
        fire_idx(c0, c0)
    fire_gathers(0, 0)
    fire_gathers(1, 1)

    step(0, 0, False, True, True)
    step(1, 1, False, True, True)
    step(2, 2, True, True, True)
    step(3, 3, True, True, True)

    def quad(i, carry):
        for k in range(NBUF):
            step(i * NBUF + k, k, True, True, True)
        return carry

    lax.fori_loop(1, NUM_CHUNKS // NBUF - 1, quad, 0)

    step(NUM_CHUNKS - 4, 0, True, True, False)
    step(NUM_CHUNKS - 3, 1, True, True, False)
    step(NUM_CHUNKS - 2, 2, True, False, False)
    step(NUM_CHUNKS - 1, 3, True, False, False)

    wait_wb()
    wait_wb()


def _sc_gather_add(flat_idx, word_table, pos_flat):
    mesh = plsc.VectorSubcoreMesh(core_axis_name="c", subcore_axis_name="s")
    return pl.kernel(
        _sc_body,
        mesh=mesh,
        compiler_params=pltpu.CompilerParams(use_tc_tiling_on_sc=False),
        out_type=jax.ShapeDtypeStruct((BATCH * SEQ_LEN, D), jnp.float32),
        scratch_types=[
            pltpu.VMEM((NBUF * CHUNK_ROWS,), jnp.int32),
            pltpu.VMEM((NBUF * CHUNK_ROWS, D), jnp.float32),
            pltpu.VMEM((SEQ_LEN * D,), jnp.float32),
            pltpu.SemaphoreType.DMA,
            pltpu.SemaphoreType.DMA,
            pltpu.SemaphoreType.DMA,
        ],
    )(flat_idx, word_table, pos_flat)


@jax.jit
def _pos_embed(flat_idx, word_table, pos_flat):
    return _sc_gather_add(flat_idx, word_table, pos_flat)


def kernel(inputs, word_table, pos_table):
    flat_idx = inputs.reshape(-1)
    pos_flat = pos_table.reshape(-1)
    out = _pos_embed(flat_idx, word_table, pos_flat)
    return out.reshape(BATCH, SEQ_LEN, D)
